# baseline (speedup 1.0000x reference)
import functools
import math

import jax
import jax.numpy as jnp
from jax import lax
from jax.experimental import pallas as pl
from jax.experimental.pallas import tpu as pltpu
from jax.experimental.pallas import tpu_sc as plsc

B, D, L_H = 4096, 128, 20
NC, NS, LANES = 2, 16, 16
NW = NC * NS
UPW = B // NW
CHUNK = 4
NCHUNK = UPW // CHUNK
CL = CHUNK * L_H
NG = D // LANES
KBUF = 4


def _sc_knn_body(table_hbm, hist_hbm, nodes_hbm, out_hbm,
                 histt_v, hist_v, nidx_v, new_v, old_v, res_v, sem_n, *sems):
    wid = lax.axis_index("s") * NC + lax.axis_index("c")
    pltpu.sync_copy(hist_hbm.at[:, pl.ds(wid * UPW, UPW)], histt_v)
    pltpu.sync_copy(nodes_hbm.at[wid], nidx_v)
    lane = lax.iota(jnp.int32, LANES)
    packs = [((lane + g * LANES) // L_H, (lane + g * LANES) % L_H)
             for g in range(CL // LANES)]

    def repack_chunk(ci):
        for g, (uvec, lvec) in enumerate(packs):
            vals = plsc.load_gather(histt_v, [lvec, ci * CHUNK + uvec])
            hist_v[ci, pl.ds(g * LANES, LANES)] = vals

    for b in range(KBUF):
        repack_chunk(b)
        pltpu.async_copy(table_hbm.at[hist_v.at[b]], old_v.at[b], sems[b])
    pltpu.async_copy(table_hbm.at[nidx_v], new_v, sem_n).wait()

    def compute_chunk(ci, old_v):
        def u_body(u, carry):
            urow = ci * CHUNK + u
            nvecs = [new_v[urow, pl.ds(j * LANES, LANES)] for j in range(NG)]
            d2s = []
            for l in range(L_H):
                row = u * L_H + l
                acc = None
                for j in range(NG):
                    dlt = old_v[row, pl.ds(j * LANES, LANES)] - nvecs[j]
                    sq = dlt * dlt
                    acc = sq if acc is None else acc + sq
                d2s.append(jnp.sum(acc))
            while len(d2s) > 1:
                d2s = [jnp.minimum(d2s[2 * i], d2s[2 * i + 1])
                       for i in range(len(d2s) // 2)] + d2s[len(d2s) & ~1:]
            lane = lax.iota(jnp.int32, LANES)
            plsc.store_scatter(res_v, [jnp.full((LANES,), urow, jnp.int32)],
                               jnp.full((LANES,), d2s[0], jnp.float32),
                               mask=lane == 0)
            return carry

        lax.fori_loop(0, CHUNK, u_body, 0)

    def group_body(g, carry):
        for b in range(KBUF):
            ci = g * KBUF + b
            pltpu.make_async_copy(table_hbm.at[hist_v.at[ci]],
                                  old_v.at[b], sems[b]).wait()
            compute_chunk(ci, old_v.at[b])

            @pl.when(ci + KBUF < NCHUNK)
            def _():
                repack_chunk(ci + KBUF)
                pltpu.async_copy(table_hbm.at[hist_v.at[ci + KBUF]],
                                 old_v.at[b], sems[b])

        return carry

    lax.fori_loop(0, NCHUNK // KBUF, group_body, 0)
    pltpu.sync_copy(res_v, out_hbm.at[pl.ds(wid * UPW, UPW)])


def _sc_knn(v_embed, hist_items, nodes_v):
    hist_r = hist_items.T
    nodes_r = nodes_v.reshape(NW, UPW)
    mesh = plsc.VectorSubcoreMesh(core_axis_name="c", subcore_axis_name="s")
    f = pl.kernel(
        _sc_knn_body,
        out_type=jax.ShapeDtypeStruct((B,), jnp.float32),
        mesh=mesh,
        compiler_params=pltpu.CompilerParams(needs_layout_passes=False),
        scratch_types=[
            pltpu.VMEM((L_H, UPW), jnp.int32),
            pltpu.VMEM((NCHUNK, CL), jnp.int32),
            pltpu.VMEM((UPW,), jnp.int32),
            pltpu.VMEM((UPW, D), jnp.float32),
            pltpu.VMEM((KBUF, CL, D), jnp.float32),
            pltpu.VMEM((UPW,), jnp.float32),
            pltpu.SemaphoreType.DMA,
        ] + [pltpu.SemaphoreType.DMA] * KBUF,
    )
    return f(v_embed, hist_r, nodes_r)


def _bn(x, g, b):
    mu = jnp.mean(x, axis=0, keepdims=True)
    var = jnp.mean((x - mu) ** 2, axis=0, keepdims=True)
    return g * (x - mu) / jnp.sqrt(var + 1e-5) + b


def _dot_t(x, w):
    return lax.dot_general(x, w, (((1,), (1,)), ((), ())),
                           preferred_element_type=jnp.float32)


_S0 = math.exp(-4) + math.exp(-3) + math.exp(-2) + math.exp(-1)
_C1, _C2 = math.exp(-4) / _S0, math.exp(-3) / _S0
_C3, _C4 = math.exp(-2) / _S0, math.exp(-1) / _S0


def _tc_head_body(e1, e2, e3, e4, ev,
                  wur1, wur2, wvr1, wvr2, wuv1a, wuv1b, wuv2, wuv3,
                  bur1, bur2, bvr1, bvr2, buv1, buv2, buv3,
                  g1, be1, g2, be2, g3, be3, g4, be4, scores_out, cn_out):
    a1, a2, a3, a4 = e1[...], e2[...], e3[...], e4[...]
    c = (jnp.sqrt(jnp.sum((a1 - a2) ** 2, axis=1, keepdims=True))
         + jnp.sqrt(jnp.sum((a2 - a3) ** 2, axis=1, keepdims=True))
         + jnp.sqrt(jnp.sum((a3 - a4) ** 2, axis=1, keepdims=True))) / 3.0
    c_lo, c_hi = jnp.min(c), jnp.max(c)
    cn_out[...] = (c - c_lo) / (c_hi - c_lo)
    u = a1 * _C1 + a2 * _C2 + a3 * _C3 + a4 * _C4
    xu = jax.nn.relu(_bn(_dot_t(u, wur1[...]) + bur1[...], g1[...], be1[...]))
    xu = _dot_t(xu, wur2[...]) + bur2[...]
    xv = jax.nn.relu(_bn(_dot_t(ev[...], wvr1[...]) + bvr1[...], g2[...], be2[...]))
    xv = _dot_t(xv, wvr2[...]) + bvr2[...]
    x = _dot_t(xu, wuv1a[...]) + _dot_t(xv, wuv1b[...]) + buv1[...]
    x = jax.nn.relu(_bn(x, g3[...], be3[...]))
    x = jax.nn.relu(_bn(_dot_t(x, wuv2[...]) + buv2[...], g4[...], be4[...]))
    scores_out[...] = jnp.sum(x * wuv3[...], axis=1, keepdims=True) + buv3[0, 0]


def _tc_combine_body(scores, cn, d2, out):
    dmin = jnp.sqrt(d2[...])
    d_lo, d_hi = jnp.min(dmin), jnp.max(dmin)
    tmp = (dmin - d_lo) / (d_hi - d_lo)
    unexp = 6.0 * tmp * jnp.exp(-6.0 * tmp)
    out[...] = scores[...] + unexp * cn[...]


def kernel(embeds_u_1, embeds_u_2, embeds_u_3, embeds_u_4, embeds_v, v_embed,
           hist_items, nodes_v,
           W_ur1, b_ur1, W_ur2, b_ur2, W_vr1, b_vr1, W_vr2, b_vr2,
           W_uv1, b_uv1, W_uv2, b_uv2, W_uv3, b_uv3,
           g1, be1, g2, be2, g3, be3, g4, be4):
    d2 = _sc_knn(v_embed, hist_items, nodes_v)
    row = lambda v: v.reshape(1, -1)
    scores, cn = pl.pallas_call(
        _tc_head_body,
        out_shape=(jax.ShapeDtypeStruct((B, 1), jnp.float32),
                   jax.ShapeDtypeStruct((B, 1), jnp.float32)),
    )(embeds_u_1, embeds_u_2, embeds_u_3, embeds_u_4, embeds_v,
      W_ur1, W_ur2, W_vr1, W_vr2, W_uv1[:, :D], W_uv1[:, D:], W_uv2, W_uv3,
      row(b_ur1), row(b_ur2), row(b_vr1), row(b_vr2),
      row(b_uv1), row(b_uv2), row(b_uv3),
      row(g1), row(be1), row(g2), row(be2), row(g3), row(be3),
      row(g4), row(be4))
    ratings = pl.pallas_call(
        _tc_combine_body,
        out_shape=jax.ShapeDtypeStruct((NW, UPW), jnp.float32),
    )(scores.reshape(NW, UPW), cn.reshape(NW, UPW), d2.reshape(NW, UPW))
    return ratings.reshape(B)


# speedup vs baseline: 1.0198x; 1.0198x over previous
import functools
import math

import jax
import jax.numpy as jnp
from jax import lax
from jax.experimental import pallas as pl
from jax.experimental.pallas import tpu as pltpu
from jax.experimental.pallas import tpu_sc as plsc

B, D, L_H = 4096, 128, 20
NC, NS, LANES = 2, 16, 16
NW = NC * NS
UPW = B // NW
CHUNK = 4
NCHUNK = UPW // CHUNK
CL = CHUNK * L_H
NG = D // LANES
KBUF = 4


def _sc_knn_body(table_hbm, hist_hbm, nodes_hbm, out_hbm,
                 hist_v, nidx_v, new_v, old_v, res_v, sem_n, sems):
    wid = lax.axis_index("s") * NC + lax.axis_index("c")
    pltpu.sync_copy(hist_hbm.at[wid], hist_v)
    pltpu.sync_copy(nodes_hbm.at[wid], nidx_v)
    for b in range(KBUF):
        pltpu.async_copy(table_hbm.at[hist_v.at[b]], old_v.at[b], sems.at[b])
    pltpu.async_copy(table_hbm.at[nidx_v], new_v, sem_n).wait()

    def compute_chunk(ci, old_v):
        def u_body(u, carry):
            urow = ci * CHUNK + u
            nvecs = [new_v[urow, pl.ds(j * LANES, LANES)] for j in range(NG)]
            d2s = []
            for l in range(L_H):
                row = u * L_H + l
                acc = None
                for j in range(NG):
                    dlt = old_v[row, pl.ds(j * LANES, LANES)] - nvecs[j]
                    sq = dlt * dlt
                    acc = sq if acc is None else acc + sq
                d2s.append(jnp.sum(acc))
            while len(d2s) > 1:
                d2s = [jnp.minimum(d2s[2 * i], d2s[2 * i + 1])
                       for i in range(len(d2s) // 2)] + d2s[len(d2s) & ~1:]
            lane = lax.iota(jnp.int32, LANES)
            plsc.store_scatter(res_v, [jnp.full((LANES,), urow, jnp.int32)],
                               jnp.full((LANES,), d2s[0], jnp.float32),
                               mask=lane == 0)
            return carry

        lax.fori_loop(0, CHUNK, u_body, 0)

    def chunk_loop(ci, carry):
        b = lax.rem(ci, KBUF)
        pltpu.make_async_copy(table_hbm.at[hist_v.at[ci]],
                              old_v.at[b], sems.at[b]).wait()
        compute_chunk(ci, old_v.at[b])

        @pl.when(ci + KBUF < NCHUNK)
        def _():
            pltpu.async_copy(table_hbm.at[hist_v.at[ci + KBUF]],
                             old_v.at[b], sems.at[b])

        return carry

    lax.fori_loop(0, NCHUNK, chunk_loop, 0)
    pltpu.sync_copy(res_v, out_hbm.at[pl.ds(wid * UPW, UPW)])


def _sc_knn(v_embed, hist_items, nodes_v):
    hist_r = hist_items.reshape(NW, NCHUNK, CL)
    nodes_r = nodes_v.reshape(NW, UPW)
    mesh = plsc.VectorSubcoreMesh(core_axis_name="c", subcore_axis_name="s")
    f = pl.kernel(
        _sc_knn_body,
        out_type=jax.ShapeDtypeStruct((B,), jnp.float32),
        mesh=mesh,
        compiler_params=pltpu.CompilerParams(needs_layout_passes=False),
        scratch_types=[
            pltpu.VMEM((NCHUNK, CL), jnp.int32),
            pltpu.VMEM((UPW,), jnp.int32),
            pltpu.VMEM((UPW, D), jnp.float32),
            pltpu.VMEM((KBUF, CL, D), jnp.float32),
            pltpu.VMEM((UPW,), jnp.float32),
            pltpu.SemaphoreType.DMA,
            pltpu.SemaphoreType.DMA((KBUF,)),
        ],
    )
    return f(v_embed, hist_r, nodes_r)


def _bn(x, g, b):
    mu = jnp.mean(x, axis=0, keepdims=True)
    var = jnp.mean((x - mu) ** 2, axis=0, keepdims=True)
    return g * (x - mu) / jnp.sqrt(var + 1e-5) + b


def _dot_t(x, w):
    return lax.dot_general(x, w, (((1,), (1,)), ((), ())),
                           preferred_element_type=jnp.float32)


_S0 = math.exp(-4) + math.exp(-3) + math.exp(-2) + math.exp(-1)
_C1, _C2 = math.exp(-4) / _S0, math.exp(-3) / _S0
_C3, _C4 = math.exp(-2) / _S0, math.exp(-1) / _S0


def _tc_head_body(e1, e2, e3, e4, ev,
                  wur1, wur2, wvr1, wvr2, wuv1a, wuv1b, wuv2, wuv3,
                  bur1, bur2, bvr1, bvr2, buv1, buv2, buv3,
                  g1, be1, g2, be2, g3, be3, g4, be4, scores_out, cn_out):
    a1, a2, a3, a4 = e1[...], e2[...], e3[...], e4[...]
    c = (jnp.sqrt(jnp.sum((a1 - a2) ** 2, axis=1, keepdims=True))
         + jnp.sqrt(jnp.sum((a2 - a3) ** 2, axis=1, keepdims=True))
         + jnp.sqrt(jnp.sum((a3 - a4) ** 2, axis=1, keepdims=True))) / 3.0
    c_lo, c_hi = jnp.min(c), jnp.max(c)
    cn_out[...] = (c - c_lo) / (c_hi - c_lo)
    u = a1 * _C1 + a2 * _C2 + a3 * _C3 + a4 * _C4
    xu = jax.nn.relu(_bn(_dot_t(u, wur1[...]) + bur1[...], g1[...], be1[...]))
    xu = _dot_t(xu, wur2[...]) + bur2[...]
    xv = jax.nn.relu(_bn(_dot_t(ev[...], wvr1[...]) + bvr1[...], g2[...], be2[...]))
    xv = _dot_t(xv, wvr2[...]) + bvr2[...]
    x = _dot_t(xu, wuv1a[...]) + _dot_t(xv, wuv1b[...]) + buv1[...]
    x = jax.nn.relu(_bn(x, g3[...], be3[...]))
    x = jax.nn.relu(_bn(_dot_t(x, wuv2[...]) + buv2[...], g4[...], be4[...]))
    scores_out[...] = jnp.sum(x * wuv3[...], axis=1, keepdims=True) + buv3[0, 0]


def _tc_combine_body(scores, cn, d2, out):
    dmin = jnp.sqrt(d2[...])
    d_lo, d_hi = jnp.min(dmin), jnp.max(dmin)
    tmp = (dmin - d_lo) / (d_hi - d_lo)
    unexp = 6.0 * tmp * jnp.exp(-6.0 * tmp)
    out[...] = scores[...] + unexp * cn[...]


def kernel(embeds_u_1, embeds_u_2, embeds_u_3, embeds_u_4, embeds_v, v_embed,
           hist_items, nodes_v,
           W_ur1, b_ur1, W_ur2, b_ur2, W_vr1, b_vr1, W_vr2, b_vr2,
           W_uv1, b_uv1, W_uv2, b_uv2, W_uv3, b_uv3,
           g1, be1, g2, be2, g3, be3, g4, be4):
    d2 = _sc_knn(v_embed, hist_items, nodes_v)
    row = lambda v: v.reshape(1, -1)
    scores, cn = pl.pallas_call(
        _tc_head_body,
        out_shape=(jax.ShapeDtypeStruct((B, 1), jnp.float32),
                   jax.ShapeDtypeStruct((B, 1), jnp.float32)),
    )(embeds_u_1, embeds_u_2, embeds_u_3, embeds_u_4, embeds_v,
      W_ur1, W_ur2, W_vr1, W_vr2, W_uv1[:, :D], W_uv1[:, D:], W_uv2, W_uv3,
      row(b_ur1), row(b_ur2), row(b_vr1), row(b_vr2),
      row(b_uv1), row(b_uv2), row(b_uv3),
      row(g1), row(be1), row(g2), row(be2), row(g3), row(be3),
      row(g4), row(be4))
    ratings = pl.pallas_call(
        _tc_combine_body,
        out_shape=jax.ShapeDtypeStruct((NW, UPW), jnp.float32),
    )(scores.reshape(NW, UPW), cn.reshape(NW, UPW), d2.reshape(NW, UPW))
    return ratings.reshape(B)


# speedup vs baseline: 1.0226x; 1.0027x over previous
import functools
import math

import jax
import jax.numpy as jnp
from jax import lax
from jax.experimental import pallas as pl
from jax.experimental.pallas import tpu as pltpu
from jax.experimental.pallas import tpu_sc as plsc

B, D, L_H = 4096, 128, 20
NC, NS, LANES = 2, 16, 16
NW = NC * NS
UPW = B // NW
CHUNK = 4
NCHUNK = UPW // CHUNK
CL = CHUNK * L_H
NG = D // LANES
KBUF = 8


def _sc_knn_body(table_hbm, hist_hbm, nodes_hbm, out_hbm,
                 hist_v, nidx_v, new_v, old_v, res_v, sem_n, sems):
    wid = lax.axis_index("s") * NC + lax.axis_index("c")
    pltpu.sync_copy(hist_hbm.at[wid], hist_v)
    pltpu.sync_copy(nodes_hbm.at[wid], nidx_v)
    for b in range(KBUF):
        pltpu.async_copy(table_hbm.at[hist_v.at[b]], old_v.at[b], sems.at[b])
    pltpu.async_copy(table_hbm.at[nidx_v], new_v, sem_n).wait()

    def compute_chunk(ci, old_v):
        def u_body(u, carry):
            urow = ci * CHUNK + u
            nvecs = [new_v[urow, pl.ds(j * LANES, LANES)] for j in range(NG)]
            d2s = []
            for l in range(L_H):
                row = u * L_H + l
                acc = None
                for j in range(NG):
                    dlt = old_v[row, pl.ds(j * LANES, LANES)] - nvecs[j]
                    sq = dlt * dlt
                    acc = sq if acc is None else acc + sq
                d2s.append(jnp.sum(acc))
            while len(d2s) > 1:
                d2s = [jnp.minimum(d2s[2 * i], d2s[2 * i + 1])
                       for i in range(len(d2s) // 2)] + d2s[len(d2s) & ~1:]
            lane = lax.iota(jnp.int32, LANES)
            plsc.store_scatter(res_v, [jnp.full((LANES,), urow, jnp.int32)],
                               jnp.full((LANES,), d2s[0], jnp.float32),
                               mask=lane == 0)
            return carry

        lax.fori_loop(0, CHUNK, u_body, 0)

    def chunk_loop(ci, carry):
        b = lax.rem(ci, KBUF)
        pltpu.make_async_copy(table_hbm.at[hist_v.at[ci]],
                              old_v.at[b], sems.at[b]).wait()
        compute_chunk(ci, old_v.at[b])

        @pl.when(ci + KBUF < NCHUNK)
        def _():
            pltpu.async_copy(table_hbm.at[hist_v.at[ci + KBUF]],
                             old_v.at[b], sems.at[b])

        return carry

    lax.fori_loop(0, NCHUNK, chunk_loop, 0)
    pltpu.sync_copy(res_v, out_hbm.at[pl.ds(wid * UPW, UPW)])


def _sc_knn(v_embed, hist_items, nodes_v):
    hist_r = hist_items.reshape(NW, NCHUNK, CL)
    nodes_r = nodes_v.reshape(NW, UPW)
    mesh = plsc.VectorSubcoreMesh(core_axis_name="c", subcore_axis_name="s")
    f = pl.kernel(
        _sc_knn_body,
        out_type=jax.ShapeDtypeStruct((B,), jnp.float32),
        mesh=mesh,
        compiler_params=pltpu.CompilerParams(needs_layout_passes=False),
        scratch_types=[
            pltpu.VMEM((NCHUNK, CL), jnp.int32),
            pltpu.VMEM((UPW,), jnp.int32),
            pltpu.VMEM((UPW, D), jnp.float32),
            pltpu.VMEM((KBUF, CL, D), jnp.float32),
            pltpu.VMEM((UPW,), jnp.float32),
            pltpu.SemaphoreType.DMA,
            pltpu.SemaphoreType.DMA((KBUF,)),
        ],
    )
    return f(v_embed, hist_r, nodes_r)


def _bn(x, g, b):
    mu = jnp.mean(x, axis=0, keepdims=True)
    var = jnp.mean((x - mu) ** 2, axis=0, keepdims=True)
    return g * (x - mu) / jnp.sqrt(var + 1e-5) + b


def _dot_t(x, w):
    return lax.dot_general(x, w, (((1,), (1,)), ((), ())),
                           preferred_element_type=jnp.float32)


_S0 = math.exp(-4) + math.exp(-3) + math.exp(-2) + math.exp(-1)
_C1, _C2 = math.exp(-4) / _S0, math.exp(-3) / _S0
_C3, _C4 = math.exp(-2) / _S0, math.exp(-1) / _S0


def _tc_head_body(e1, e2, e3, e4, ev,
                  wur1, wur2, wvr1, wvr2, wuv1a, wuv1b, wuv2, wuv3,
                  bur1, bur2, bvr1, bvr2, buv1, buv2, buv3,
                  g1, be1, g2, be2, g3, be3, g4, be4, scores_out, cn_out):
    a1, a2, a3, a4 = e1[...], e2[...], e3[...], e4[...]
    c = (jnp.sqrt(jnp.sum((a1 - a2) ** 2, axis=1, keepdims=True))
         + jnp.sqrt(jnp.sum((a2 - a3) ** 2, axis=1, keepdims=True))
         + jnp.sqrt(jnp.sum((a3 - a4) ** 2, axis=1, keepdims=True))) / 3.0
    c_lo, c_hi = jnp.min(c), jnp.max(c)
    cn_out[...] = (c - c_lo) / (c_hi - c_lo)
    u = a1 * _C1 + a2 * _C2 + a3 * _C3 + a4 * _C4
    xu = jax.nn.relu(_bn(_dot_t(u, wur1[...]) + bur1[...], g1[...], be1[...]))
    xu = _dot_t(xu, wur2[...]) + bur2[...]
    xv = jax.nn.relu(_bn(_dot_t(ev[...], wvr1[...]) + bvr1[...], g2[...], be2[...]))
    xv = _dot_t(xv, wvr2[...]) + bvr2[...]
    x = _dot_t(xu, wuv1a[...]) + _dot_t(xv, wuv1b[...]) + buv1[...]
    x = jax.nn.relu(_bn(x, g3[...], be3[...]))
    x = jax.nn.relu(_bn(_dot_t(x, wuv2[...]) + buv2[...], g4[...], be4[...]))
    scores_out[...] = jnp.sum(x * wuv3[...], axis=1, keepdims=True) + buv3[0, 0]


def _tc_combine_body(scores, cn, d2, out):
    dmin = jnp.sqrt(d2[...])
    d_lo, d_hi = jnp.min(dmin), jnp.max(dmin)
    tmp = (dmin - d_lo) / (d_hi - d_lo)
    unexp = 6.0 * tmp * jnp.exp(-6.0 * tmp)
    out[...] = scores[...] + unexp * cn[...]


def kernel(embeds_u_1, embeds_u_2, embeds_u_3, embeds_u_4, embeds_v, v_embed,
           hist_items, nodes_v,
           W_ur1, b_ur1, W_ur2, b_ur2, W_vr1, b_vr1, W_vr2, b_vr2,
           W_uv1, b_uv1, W_uv2, b_uv2, W_uv3, b_uv3,
           g1, be1, g2, be2, g3, be3, g4, be4):
    d2 = _sc_knn(v_embed, hist_items, nodes_v)
    row = lambda v: v.reshape(1, -1)
    scores, cn = pl.pallas_call(
        _tc_head_body,
        out_shape=(jax.ShapeDtypeStruct((B, 1), jnp.float32),
                   jax.ShapeDtypeStruct((B, 1), jnp.float32)),
    )(embeds_u_1, embeds_u_2, embeds_u_3, embeds_u_4, embeds_v,
      W_ur1, W_ur2, W_vr1, W_vr2, W_uv1[:, :D], W_uv1[:, D:], W_uv2, W_uv3,
      row(b_ur1), row(b_ur2), row(b_vr1), row(b_vr2),
      row(b_uv1), row(b_uv2), row(b_uv3),
      row(g1), row(be1), row(g2), row(be2), row(g3), row(be3),
      row(g4), row(be4))
    ratings = pl.pallas_call(
        _tc_combine_body,
        out_shape=jax.ShapeDtypeStruct((NW, UPW), jnp.float32),
    )(scores.reshape(NW, UPW), cn.reshape(NW, UPW), d2.reshape(NW, UPW))
    return ratings.reshape(B)


# speedup vs baseline: 1.0293x; 1.0065x over previous
import math

import jax
import jax.numpy as jnp
from jax import lax
from jax.experimental import pallas as pl
from jax.experimental.pallas import tpu as pltpu
from jax.experimental.pallas import tpu_sc as plsc

B, D, L_H = 4096, 128, 20
NC, NS, LANES = 2, 16, 16
NW = NC * NS
UPW = B // NW
CHUNK = 4
NCHUNK = UPW // CHUNK
CL = CHUNK * L_H
NG = D // LANES
KBUF = 8


def _sc_knn_body(table_hbm, hist_hbm, nodes_hbm, out_hbm,
                 hist_v, nidx_v, new_v, old_v, res_v, sem_n, sems):
    wid = lax.axis_index("s") * NC + lax.axis_index("c")
    pltpu.sync_copy(hist_hbm.at[wid], hist_v)
    pltpu.sync_copy(nodes_hbm.at[wid], nidx_v)
    for b in range(KBUF):
        pltpu.async_copy(table_hbm.at[hist_v.at[b]], old_v.at[b], sems.at[b])
    pltpu.async_copy(table_hbm.at[nidx_v], new_v, sem_n).wait()

    def compute_chunk(ci, old_ref):
        def u_body(u, carry):
            urow = ci * CHUNK + u
            nvecs = [new_v[urow, pl.ds(j * LANES, LANES)] for j in range(NG)]
            d2s = []
            for l in range(L_H):
                row = u * L_H + l
                acc = None
                for j in range(NG):
                    dlt = old_ref[row, pl.ds(j * LANES, LANES)] - nvecs[j]
                    sq = dlt * dlt
                    acc = sq if acc is None else acc + sq
                d2s.append(jnp.sum(acc))
            while len(d2s) > 1:
                d2s = [jnp.minimum(d2s[2 * i], d2s[2 * i + 1])
                       for i in range(len(d2s) // 2)] + d2s[len(d2s) & ~1:]
            lane = lax.iota(jnp.int32, LANES)
            plsc.store_scatter(res_v, [jnp.full((LANES,), urow, jnp.int32)],
                               jnp.full((LANES,), d2s[0], jnp.float32),
                               mask=lane == 0)
            return carry

        lax.fori_loop(0, CHUNK, u_body, 0)

    def chunk_loop(ci, carry):
        b = lax.rem(ci, KBUF)
        pltpu.make_async_copy(table_hbm.at[hist_v.at[ci]],
                              old_v.at[b], sems.at[b]).wait()
        compute_chunk(ci, old_v.at[b])

        @pl.when(ci + KBUF < NCHUNK)
        def _():
            pltpu.async_copy(table_hbm.at[hist_v.at[ci + KBUF]],
                             old_v.at[b], sems.at[b])

        return carry

    lax.fori_loop(0, NCHUNK, chunk_loop, 0)
    pltpu.sync_copy(res_v, out_hbm.at[pl.ds(wid * UPW, UPW)])


def _sc_knn(v_embed, hist_items, nodes_v):
    hist_r = hist_items.reshape(NW, NCHUNK, CL)
    nodes_r = nodes_v.reshape(NW, UPW)
    mesh = plsc.VectorSubcoreMesh(core_axis_name="c", subcore_axis_name="s")
    f = pl.kernel(
        _sc_knn_body,
        out_type=jax.ShapeDtypeStruct((B,), jnp.float32),
        mesh=mesh,
        compiler_params=pltpu.CompilerParams(needs_layout_passes=False),
        scratch_types=[
            pltpu.VMEM((NCHUNK, CL), jnp.int32),
            pltpu.VMEM((UPW,), jnp.int32),
            pltpu.VMEM((UPW, D), jnp.float32),
            pltpu.VMEM((KBUF, CL, D), jnp.float32),
            pltpu.VMEM((UPW,), jnp.float32),
            pltpu.SemaphoreType.DMA,
            pltpu.SemaphoreType.DMA((KBUF,)),
        ],
    )
    return f(v_embed, hist_r, nodes_r)


def _bn(x, g, b):
    mu = jnp.mean(x, axis=0, keepdims=True)
    var = jnp.mean((x - mu) ** 2, axis=0, keepdims=True)
    return g * (x - mu) / jnp.sqrt(var + 1e-5) + b


def _dot_t(x, w):
    return lax.dot_general(x, w, (((1,), (1,)), ((), ())),
                           preferred_element_type=jnp.float32)


_S0 = math.exp(-4) + math.exp(-3) + math.exp(-2) + math.exp(-1)
_C1, _C2 = math.exp(-4) / _S0, math.exp(-3) / _S0
_C3, _C4 = math.exp(-2) / _S0, math.exp(-1) / _S0


def _tc_head_body(e1, e2, e3, e4, ev,
                  wur1, wur2, wvr1, wvr2, wuv1a, wuv1b, wuv2, wuv3,
                  bur1, bur2, bvr1, bvr2, buv1, buv2, buv3,
                  g1, be1, g2, be2, g3, be3, g4, be4, scores_out, cn_out):
    a1, a2, a3, a4 = e1[...], e2[...], e3[...], e4[...]
    c = (jnp.sqrt(jnp.sum((a1 - a2) ** 2, axis=1, keepdims=True))
         + jnp.sqrt(jnp.sum((a2 - a3) ** 2, axis=1, keepdims=True))
         + jnp.sqrt(jnp.sum((a3 - a4) ** 2, axis=1, keepdims=True))) / 3.0
    c_lo, c_hi = jnp.min(c), jnp.max(c)
    cn_out[...] = (c - c_lo) / (c_hi - c_lo)
    u = a1 * _C1 + a2 * _C2 + a3 * _C3 + a4 * _C4
    xu = jax.nn.relu(_bn(_dot_t(u, wur1[...]) + bur1[...], g1[...], be1[...]))
    xu = _dot_t(xu, wur2[...]) + bur2[...]
    xv = jax.nn.relu(_bn(_dot_t(ev[...], wvr1[...]) + bvr1[...], g2[...], be2[...]))
    xv = _dot_t(xv, wvr2[...]) + bvr2[...]
    x = _dot_t(xu, wuv1a[...]) + _dot_t(xv, wuv1b[...]) + buv1[...]
    x = jax.nn.relu(_bn(x, g3[...], be3[...]))
    x = jax.nn.relu(_bn(_dot_t(x, wuv2[...]) + buv2[...], g4[...], be4[...]))
    scores_out[...] = jnp.sum(x * wuv3[...], axis=1, keepdims=True) + buv3[0, 0]


def _tc_combine_body(scores, cn, d2, out):
    dmin = jnp.sqrt(d2[...])
    d_lo, d_hi = jnp.min(dmin), jnp.max(dmin)
    tmp = (dmin - d_lo) / (d_hi - d_lo)
    unexp = 6.0 * tmp * jnp.exp(-6.0 * tmp)
    out[...] = scores[...] + unexp * cn[...]


def kernel(embeds_u_1, embeds_u_2, embeds_u_3, embeds_u_4, embeds_v, v_embed,
           hist_items, nodes_v,
           W_ur1, b_ur1, W_ur2, b_ur2, W_vr1, b_vr1, W_vr2, b_vr2,
           W_uv1, b_uv1, W_uv2, b_uv2, W_uv3, b_uv3,
           g1, be1, g2, be2, g3, be3, g4, be4):
    d2 = _sc_knn(v_embed, hist_items, nodes_v)
    row = lambda v: v.reshape(1, -1)
    scores, cn = pl.pallas_call(
        _tc_head_body,
        out_shape=(jax.ShapeDtypeStruct((B, 1), jnp.float32),
                   jax.ShapeDtypeStruct((B, 1), jnp.float32)),
    )(embeds_u_1, embeds_u_2, embeds_u_3, embeds_u_4, embeds_v,
      W_ur1, W_ur2, W_vr1, W_vr2, W_uv1[:, :D], W_uv1[:, D:], W_uv2, W_uv3,
      row(b_ur1), row(b_ur2), row(b_vr1), row(b_vr2),
      row(b_uv1), row(b_uv2), row(b_uv3),
      row(g1), row(be1), row(g2), row(be2), row(g3), row(be3),
      row(g4), row(be4))
    ratings = pl.pallas_call(
        _tc_combine_body,
        out_shape=jax.ShapeDtypeStruct((NW, UPW), jnp.float32),
    )(scores.reshape(NW, UPW), cn.reshape(NW, UPW), d2.reshape(NW, UPW))
    return ratings.reshape(B)


# speedup vs baseline: 1.0390x; 1.0094x over previous
import math

import jax
import jax.numpy as jnp
from jax import lax
from jax.experimental import pallas as pl
from jax.experimental.pallas import tpu as pltpu
from jax.experimental.pallas import tpu_sc as plsc

B, D, L_H = 4096, 128, 20
NC, NS, LANES = 2, 16, 16
NW = NC * NS
UPW = B // NW
CHUNK = 4
NCHUNK = UPW // CHUNK
CL = CHUNK * L_H
NG = D // LANES
KBUF = 8


def _sc_knn_body(table_hbm, hist_hbm, nodes_hbm, out_hbm,
                 hist_v, nidx_v, new_v, old_v, res_v, sem_n, sems):
    wid = lax.axis_index("s") * NC + lax.axis_index("c")
    pltpu.sync_copy(nodes_hbm.at[wid], nidx_v)
    new_cp = pltpu.async_copy(table_hbm.at[nidx_v], new_v, sem_n)
    pltpu.sync_copy(hist_hbm.at[wid], hist_v)
    for b in range(KBUF):
        pltpu.async_copy(table_hbm.at[hist_v.at[b]], old_v.at[b], sems.at[b])
    new_cp.wait()

    def compute_chunk(ci, old_ref):
        def u_body(u, carry):
            urow = ci * CHUNK + u
            nvecs = [new_v[urow, pl.ds(j * LANES, LANES)] for j in range(NG)]
            d2s = []
            for l in range(L_H):
                row = u * L_H + l
                acc = None
                for j in range(NG):
                    dlt = old_ref[row, pl.ds(j * LANES, LANES)] - nvecs[j]
                    sq = dlt * dlt
                    acc = sq if acc is None else acc + sq
                d2s.append(jnp.sum(acc))
            while len(d2s) > 1:
                d2s = [jnp.minimum(d2s[2 * i], d2s[2 * i + 1])
                       for i in range(len(d2s) // 2)] + d2s[len(d2s) & ~1:]
            lane = lax.iota(jnp.int32, LANES)
            plsc.store_scatter(res_v, [jnp.full((LANES,), urow, jnp.int32)],
                               jnp.full((LANES,), d2s[0], jnp.float32),
                               mask=lane == 0)
            return carry

        lax.fori_loop(0, CHUNK, u_body, 0)

    def chunk_loop(ci, carry):
        b = lax.rem(ci, KBUF)
        pltpu.make_async_copy(table_hbm.at[hist_v.at[ci]],
                              old_v.at[b], sems.at[b]).wait()
        compute_chunk(ci, old_v.at[b])

        @pl.when(ci + KBUF < NCHUNK)
        def _():
            pltpu.async_copy(table_hbm.at[hist_v.at[ci + KBUF]],
                             old_v.at[b], sems.at[b])

        return carry

    lax.fori_loop(0, NCHUNK, chunk_loop, 0)
    pltpu.sync_copy(res_v, out_hbm.at[pl.ds(wid * UPW, UPW)])


def _sc_knn(v_embed, hist_items, nodes_v):
    hist_r = hist_items.reshape(NW, NCHUNK, CL)
    nodes_r = nodes_v.reshape(NW, UPW)
    mesh = plsc.VectorSubcoreMesh(core_axis_name="c", subcore_axis_name="s")
    f = pl.kernel(
        _sc_knn_body,
        out_type=jax.ShapeDtypeStruct((B,), jnp.float32),
        mesh=mesh,
        compiler_params=pltpu.CompilerParams(needs_layout_passes=False),
        scratch_types=[
            pltpu.VMEM((NCHUNK, CL), jnp.int32),
            pltpu.VMEM((UPW,), jnp.int32),
            pltpu.VMEM((UPW, D), jnp.float32),
            pltpu.VMEM((KBUF, CL, D), jnp.float32),
            pltpu.VMEM((UPW,), jnp.float32),
            pltpu.SemaphoreType.DMA,
            pltpu.SemaphoreType.DMA((KBUF,)),
        ],
    )
    return f(v_embed, hist_r, nodes_r)


def _bn(x, g, b):
    mu = jnp.mean(x, axis=0, keepdims=True)
    var = jnp.mean((x - mu) ** 2, axis=0, keepdims=True)
    return g * (x - mu) / jnp.sqrt(var + 1e-5) + b


def _dot_t(x, w):
    return lax.dot_general(x, w, (((1,), (1,)), ((), ())),
                           preferred_element_type=jnp.float32)


_S0 = math.exp(-4) + math.exp(-3) + math.exp(-2) + math.exp(-1)
_C1, _C2 = math.exp(-4) / _S0, math.exp(-3) / _S0
_C3, _C4 = math.exp(-2) / _S0, math.exp(-1) / _S0


def _tc_head_body(e1, e2, e3, e4, ev,
                  wur1, wur2, wvr1, wvr2, wuv1a, wuv1b, wuv2, wuv3,
                  bur1, bur2, bvr1, bvr2, buv1, buv2, buv3,
                  g1, be1, g2, be2, g3, be3, g4, be4, scores_out, cn_out):
    a1, a2, a3, a4 = e1[...], e2[...], e3[...], e4[...]
    c = (jnp.sqrt(jnp.sum((a1 - a2) ** 2, axis=1, keepdims=True))
         + jnp.sqrt(jnp.sum((a2 - a3) ** 2, axis=1, keepdims=True))
         + jnp.sqrt(jnp.sum((a3 - a4) ** 2, axis=1, keepdims=True))) / 3.0
    c_lo, c_hi = jnp.min(c), jnp.max(c)
    cn_out[...] = (c - c_lo) / (c_hi - c_lo)
    u = a1 * _C1 + a2 * _C2 + a3 * _C3 + a4 * _C4
    xu = jax.nn.relu(_bn(_dot_t(u, wur1[...]) + bur1[...], g1[...], be1[...]))
    xu = _dot_t(xu, wur2[...]) + bur2[...]
    xv = jax.nn.relu(_bn(_dot_t(ev[...], wvr1[...]) + bvr1[...], g2[...], be2[...]))
    xv = _dot_t(xv, wvr2[...]) + bvr2[...]
    x = _dot_t(xu, wuv1a[...]) + _dot_t(xv, wuv1b[...]) + buv1[...]
    x = jax.nn.relu(_bn(x, g3[...], be3[...]))
    x = jax.nn.relu(_bn(_dot_t(x, wuv2[...]) + buv2[...], g4[...], be4[...]))
    scores_out[...] = jnp.sum(x * wuv3[...], axis=1, keepdims=True) + buv3[0, 0]


def _tc_combine_body(scores, cn, d2, out):
    dmin = jnp.sqrt(d2[...])
    d_lo, d_hi = jnp.min(dmin), jnp.max(dmin)
    tmp = (dmin - d_lo) / (d_hi - d_lo)
    unexp = 6.0 * tmp * jnp.exp(-6.0 * tmp)
    out[...] = scores[...] + unexp * cn[...]


def kernel(embeds_u_1, embeds_u_2, embeds_u_3, embeds_u_4, embeds_v, v_embed,
           hist_items, nodes_v,
           W_ur1, b_ur1, W_ur2, b_ur2, W_vr1, b_vr1, W_vr2, b_vr2,
           W_uv1, b_uv1, W_uv2, b_uv2, W_uv3, b_uv3,
           g1, be1, g2, be2, g3, be3, g4, be4):
    d2 = _sc_knn(v_embed, hist_items, nodes_v)
    row = lambda v: v.reshape(1, -1)
    scores, cn = pl.pallas_call(
        _tc_head_body,
        out_shape=(jax.ShapeDtypeStruct((B, 1), jnp.float32),
                   jax.ShapeDtypeStruct((B, 1), jnp.float32)),
    )(embeds_u_1, embeds_u_2, embeds_u_3, embeds_u_4, embeds_v,
      W_ur1, W_ur2, W_vr1, W_vr2, W_uv1[:, :D], W_uv1[:, D:], W_uv2, W_uv3,
      row(b_ur1), row(b_ur2), row(b_vr1), row(b_vr2),
      row(b_uv1), row(b_uv2), row(b_uv3),
      row(g1), row(be1), row(g2), row(be2), row(g3), row(be3),
      row(g4), row(be4))
    ratings = pl.pallas_call(
        _tc_combine_body,
        out_shape=jax.ShapeDtypeStruct((NW, UPW), jnp.float32),
    )(scores.reshape(NW, UPW), cn.reshape(NW, UPW), d2.reshape(NW, UPW))
    return ratings.reshape(B)
